# Initial kernel scaffold; baseline (speedup 1.0000x reference)
import jax
import jax.numpy as jnp
from jax.experimental import pallas as pl


def kernel(h_user, h_item, edge_src, edge_dst, dt, norm, dt_r, norm_r, decay_lam):
    raise NotImplementedError("write your pallas kernel here")


# speedup vs baseline: 2.3253x; 2.3253x over previous
import functools

import jax
import jax.numpy as jnp
from jax import lax
from jax.experimental import pallas as pl
from jax.experimental.pallas import tpu as pltpu
from jax.experimental.pallas import tpu_sc as plsc

N_USER = 5000
N_ITEM = 5000
E = 320000
D = 128

NC = 2
NS = 16
L = 16
NW = NC * NS

K = 80
PER_TILE = E // NW
N_CHUNKS = PER_TILE // K

ZROWS = 200
N_STRIPES = N_USER // ZROWS


def _sc_body(h_user, h_item, src_hbm, dst_hbm, dt_hbm, nm_hbm, dtr_hbm,
             nmr_hbm, nlam_hbm, pu_out, pi_out,
             src_v, dst_v, dt_v, nm_v, dtr_v, nmr_v, w_ui, w_iu,
             rows_ui, rows_iu, zbuf, lam_v, acc_user, acc_item, sem1, sem2):
  c = lax.axis_index("c")
  s = lax.axis_index("s")
  wid = c * NS + s

  def zero_row(r, carry):
    for j in range(D // L):
      zbuf[r, pl.ds(j * L, L)] = jnp.zeros((L,), jnp.float32)
    return carry
  lax.fori_loop(0, ZROWS, zero_row, 0)

  for j in range(2):
    stripe = s + NS * j
    @pl.when(stripe < N_STRIPES)
    def _():
      pltpu.sync_copy(zbuf, acc_user.at[pl.ds(stripe * ZROWS, ZROWS)])
      pltpu.sync_copy(zbuf, acc_item.at[pl.ds(stripe * ZROWS, ZROWS)])

  pltpu.sync_copy(nlam_hbm, lam_v)
  nlam = lam_v[...]

  plsc.subcore_barrier()

  base = wid * PER_TILE

  def chunk_body(jc, carry):
    e0 = base + jc * K
    pltpu.sync_copy(src_hbm.at[pl.ds(e0, K)], src_v)
    pltpu.sync_copy(dst_hbm.at[pl.ds(e0, K)], dst_v)
    g1 = pltpu.async_copy(h_user.at[src_v], rows_ui, sem1)
    g2 = pltpu.async_copy(h_item.at[dst_v], rows_iu, sem2)
    pltpu.sync_copy(dt_hbm.at[pl.ds(e0, K)], dt_v)
    pltpu.sync_copy(nm_hbm.at[pl.ds(e0, K)], nm_v)
    pltpu.sync_copy(dtr_hbm.at[pl.ds(e0, K)], dtr_v)
    pltpu.sync_copy(nmr_hbm.at[pl.ds(e0, K)], nmr_v)
    for i in range(K // L):
      sl = pl.ds(i * L, L)
      w_ui[sl] = nm_v[sl] * jnp.exp(nlam * dt_v[sl])
      w_iu[sl] = nmr_v[sl] * jnp.exp(nlam * dtr_v[sl])
    g1.wait()
    g2.wait()

    def scale_edge(e, carry2):
      spl = jnp.full((L,), e, jnp.int32)
      s_ui = plsc.load_gather(w_ui, [spl])
      s_iu = plsc.load_gather(w_iu, [spl])
      for j in range(D // L):
        sl2 = pl.ds(j * L, L)
        rows_ui[e, sl2] = rows_ui[e, sl2] * s_ui
        rows_iu[e, sl2] = rows_iu[e, sl2] * s_iu
      return carry2
    lax.fori_loop(0, K, scale_edge, 0)

    pltpu.sync_copy(rows_ui, acc_item.at[dst_v], add=True)
    pltpu.sync_copy(rows_iu, acc_user.at[src_v], add=True)
    return carry

  lax.fori_loop(0, N_CHUNKS, chunk_body, 0)

  plsc.subcore_barrier()

  for j in range(2):
    stripe = s + NS * j
    @pl.when(stripe < N_STRIPES)
    def _():
      sl = pl.ds(stripe * ZROWS, ZROWS)
      pltpu.sync_copy(acc_user.at[sl], pu_out.at[c, sl])
      pltpu.sync_copy(acc_item.at[sl], pi_out.at[c, sl])


_sc_call = functools.partial(
    pl.kernel,
    out_type=(
        jax.ShapeDtypeStruct((NC, N_USER, D), jnp.float32),
        jax.ShapeDtypeStruct((NC, N_ITEM, D), jnp.float32),
    ),
    mesh=plsc.VectorSubcoreMesh(
        core_axis_name="c", subcore_axis_name="s",
        num_cores=NC, num_subcores=NS),
    compiler_params=pltpu.CompilerParams(needs_layout_passes=False),
    scratch_types=[
        pltpu.VMEM((K,), jnp.int32),
        pltpu.VMEM((K,), jnp.int32),
        pltpu.VMEM((K,), jnp.float32),
        pltpu.VMEM((K,), jnp.float32),
        pltpu.VMEM((K,), jnp.float32),
        pltpu.VMEM((K,), jnp.float32),
        pltpu.VMEM((K,), jnp.float32),
        pltpu.VMEM((K,), jnp.float32),
        pltpu.VMEM((K, D), jnp.float32),
        pltpu.VMEM((K, D), jnp.float32),
        pltpu.VMEM((ZROWS, D), jnp.float32),
        pltpu.VMEM((L,), jnp.float32),
        pltpu.VMEM_SHARED((N_USER, D), jnp.float32),
        pltpu.VMEM_SHARED((N_ITEM, D), jnp.float32),
        pltpu.SemaphoreType.DMA,
        pltpu.SemaphoreType.DMA,
    ],
)(_sc_body)


def _combine_body(pu_ref, pi_ref, ou_ref, oi_ref):
  ou_ref[...] = pu_ref[0] + pu_ref[1]
  oi_ref[...] = pi_ref[0] + pi_ref[1]


def kernel(h_user, h_item, edge_src, edge_dst, dt, norm, dt_r, norm_r,
           decay_lam):
  neg_lam = -(jax.nn.relu(decay_lam.astype(jnp.float32)) + 0.0001)
  nl16 = jnp.full((L,), neg_lam, jnp.float32)
  src = edge_src.astype(jnp.int32)
  dst = edge_dst.astype(jnp.int32)
  pu, pi = _sc_call(h_user, h_item, src, dst, dt, norm, dt_r, norm_r, nl16)
  hu, hi = pl.pallas_call(
      _combine_body,
      out_shape=(
          jax.ShapeDtypeStruct((N_USER, D), jnp.float32),
          jax.ShapeDtypeStruct((N_ITEM, D), jnp.float32),
      ),
  )(pu, pi)
  return hu, hi
